# baseline (speedup 1.0000x reference)
import functools
import jax
import jax.numpy as jnp
from jax import lax
from jax.experimental import pallas as pl
from jax.experimental.pallas import tpu as pltpu
from jax.experimental.pallas import tpu_sc as plsc

B = 4096
H = 20
D = 300
V = 100000
L = 16
NC = 2
NS = 16
NW = NC * NS
ROWS_PER_W = B // NW
G = 4
NG = ROWS_PER_W // G
NCHUNK = (D + L - 1) // L

RW = 64
RPE = 6
M = V * D // RW
IPR = H * RPE

_mesh = plsc.VectorSubcoreMesh(core_axis_name="c", subcore_axis_name="s")


@functools.partial(
    pl.kernel,
    mesh=_mesh,
    compiler_params=pltpu.CompilerParams(use_tc_tiling_on_sc=False),
    out_type=jax.ShapeDtypeStruct((B, D), jnp.float32),
    scratch_types=[
        pltpu.VMEM((G * IPR,), jnp.int32),
        pltpu.VMEM((G * H,), jnp.int32),
        pltpu.VMEM((G * H,), jnp.float32),
        pltpu.VMEM((G * IPR, RW), jnp.float32),
        pltpu.VMEM((G, D), jnp.float32),
        pltpu.SemaphoreType.DMA,
    ],
)
def _embed_sum(idx2_hbm, s_hbm, w_hbm, table_hbm, out_hbm,
               idx_v, s_v, w_v, rows_v, out_v, sem):
    wid = lax.axis_index("s") * NC + lax.axis_index("c")
    base0 = wid * ROWS_PER_W

    def body(g, carry):
        base = base0 + g * G
        pltpu.sync_copy(idx2_hbm.at[pl.ds(base * IPR, G * IPR)], idx_v)
        pltpu.sync_copy(s_hbm.at[pl.ds(base * H, G * H)], s_v)
        pltpu.sync_copy(w_hbm.at[pl.ds(base * H, G * H)], w_v)
        copies = [
            pltpu.async_copy(
                table_hbm.at[idx_v.at[pl.ds(r * IPR, IPR)]],
                rows_v.at[pl.ds(r * IPR, IPR)],
                sem,
            )
            for r in range(G)
        ]
        for c in copies:
            c.wait()
        for r in range(G):
            wa = w_v[pl.ds(r * H, L)]
            wb = w_v[pl.ds(r * H + H - L, L)]
            wvs = [wa[h] if h < L else wb[h - (H - L)] for h in range(H)]
            sa = s_v[pl.ds(r * H, L)]
            sb = s_v[pl.ds(r * H + H - L, L)]
            svs = [sa[h] if h < L else sb[h - (H - L)] for h in range(H)]
            for c in range(NCHUNK):
                off = min(c * L, D - L)
                acc = wvs[0] * rows_v[r * IPR, pl.ds(svs[0] + off, L)]
                for h in range(1, H):
                    acc = acc + wvs[h] * rows_v[r * IPR + h * RPE,
                                                pl.ds(svs[h] + off, L)]
                out_v[r, pl.ds(off, L)] = acc
        pltpu.sync_copy(out_v, out_hbm.at[pl.ds(base, G)])
        return carry

    lax.fori_loop(0, NG, body, 0)


def kernel(batch_hnym, batch_hnym_weights, table):
    idx = batch_hnym.reshape(-1).astype(jnp.int32)
    word0 = idx * D
    r0 = word0 // RW
    idx2 = jnp.minimum(r0[:, None] + jnp.arange(RPE, dtype=jnp.int32), M - 1)
    s = word0 % RW
    w = batch_hnym_weights.reshape(-1)
    table_r = table.reshape(M, RW)
    return _embed_sum(idx2.reshape(-1), s, w, table_r)


# speedup vs baseline: 2.5492x; 2.5492x over previous
import functools
import jax
import jax.numpy as jnp
from jax import lax
from jax.experimental import pallas as pl
from jax.experimental.pallas import tpu as pltpu
from jax.experimental.pallas import tpu_sc as plsc

B = 4096
H = 20
D = 300
V = 100000
L = 16
NC = 2
NS = 16
NW = NC * NS
ROWS_PER_W = B // NW
G = 4
NG = ROWS_PER_W // G
NCHUNK = (D + L - 1) // L

_mesh = plsc.VectorSubcoreMesh(core_axis_name="c", subcore_axis_name="s")


@functools.partial(
    pl.kernel,
    mesh=_mesh,
    out_type=jax.ShapeDtypeStruct((B, D), jnp.float32),
    scratch_types=[
        pltpu.VMEM((G * H,), jnp.int32),
        pltpu.VMEM((G * H,), jnp.float32),
        pltpu.VMEM((G * H, D), jnp.float32),
        pltpu.VMEM((G, D), jnp.float32),
        pltpu.SemaphoreType.DMA,
    ],
)
def _embed_sum(idx_hbm, w_hbm, table_hbm, out_hbm, idx_v, w_v, rows_v, out_v, sem):
    wid = lax.axis_index("s") * NC + lax.axis_index("c")
    base0 = wid * ROWS_PER_W

    def body(g, carry):
        base = base0 + g * G
        pltpu.sync_copy(idx_hbm.at[pl.ds(base * H, G * H)], idx_v)
        pltpu.sync_copy(w_hbm.at[pl.ds(base * H, G * H)], w_v)
        ichunks = [idx_v[pl.ds(L * i, L)] for i in range(G * H // L)]
        copies = []
        for k in range(G * H):
            v = ichunks[k // L][k % L]
            copies.append(pltpu.async_copy(table_hbm.at[v], rows_v.at[k], sem))
        for cp in copies:
            cp.wait()
        for r in range(G):
            wa = w_v[pl.ds(r * H, L)]
            wb = w_v[pl.ds(r * H + H - L, L)]
            wvs = [wa[h] if h < L else wb[h - (H - L)] for h in range(H)]
            for c in range(NCHUNK):
                off = min(c * L, D - L)
                acc = wvs[0] * rows_v[r * H, pl.ds(off, L)]
                for h in range(1, H):
                    acc = acc + wvs[h] * rows_v[r * H + h, pl.ds(off, L)]
                out_v[r, pl.ds(off, L)] = acc
        pltpu.sync_copy(out_v, out_hbm.at[pl.ds(base, G)])
        return carry

    lax.fori_loop(0, NG, body, 0)


def kernel(batch_hnym, batch_hnym_weights, table):
    idx = batch_hnym.reshape(-1).astype(jnp.int32)
    w = batch_hnym_weights.reshape(-1)
    return _embed_sum(idx, w, table)


# speedup vs baseline: 2.6600x; 1.0434x over previous
import functools
import jax
import jax.numpy as jnp
from jax import lax
from jax.experimental import pallas as pl
from jax.experimental.pallas import tpu as pltpu
from jax.experimental.pallas import tpu_sc as plsc

B = 4096
H = 20
D = 300
V = 100000
L = 16
NC = 2
NS = 16
NW = NC * NS
ROWS_PER_W = B // NW
G = 4
NG = ROWS_PER_W // G
NGT = B // G
GH = G * H
IW = 2 * GH
NCHUNK = (D + L - 1) // L

_mesh = plsc.VectorSubcoreMesh(core_axis_name="c", subcore_axis_name="s")


@functools.partial(
    pl.kernel,
    mesh=_mesh,
    out_type=jax.ShapeDtypeStruct((B, D), jnp.float32),
    scratch_types=[
        pltpu.VMEM((2, GH), jnp.int32),
        pltpu.VMEM((2, GH), jnp.float32),
        pltpu.VMEM((2, GH, D), jnp.float32),
        pltpu.VMEM((2, G, D), jnp.float32),
        pltpu.SemaphoreType.DMA,
        pltpu.SemaphoreType.DMA,
        pltpu.SemaphoreType.DMA,
    ],
)
def _embed_sum(idx_hbm, w_hbm, table_hbm, out_hbm, idx_v, w_v, rows_v, out_v,
               sem0, sem1, osem):
    wid = lax.axis_index("s") * NC + lax.axis_index("c")
    g0 = wid * NG
    base0 = wid * ROWS_PER_W
    sems = [sem0, sem1]

    def load_iw(g, p):
        pltpu.sync_copy(idx_hbm.at[pl.ds(g * GH, GH)], idx_v.at[p])
        pltpu.sync_copy(w_hbm.at[pl.ds(g * GH, GH)], w_v.at[p])

    def fire(p):
        ichunks = [idx_v[p, pl.ds(L * i, L)] for i in range(GH // L)]
        for k in range(GH):
            v = ichunks[k // L][k % L]
            pltpu.async_copy(table_hbm.at[v], rows_v.at[p, k], sems[p])

    def compute(g, p):
        for k in range(GH):
            pltpu.make_async_copy(table_hbm.at[0], rows_v.at[p, k],
                                  sems[p]).wait()
        for r in range(G):
            wa = w_v[p, pl.ds(r * H, L)]
            wb = w_v[p, pl.ds(r * H + H - L, L)]
            wvs = [wa[h] if h < L else wb[h - (H - L)] for h in range(H)]
            for c in range(NCHUNK):
                off = min(c * L, D - L)
                acc = wvs[0] * rows_v[p, r * H, pl.ds(off, L)]
                for h in range(1, H):
                    acc = acc + wvs[h] * rows_v[p, r * H + h, pl.ds(off, L)]
                out_v[p, r, pl.ds(off, L)] = acc
        base = base0 + (g - g0) * G
        pltpu.async_copy(out_v.at[p], out_hbm.at[pl.ds(base, G)], osem)

    load_iw(g0, 0)
    fire(0)
    load_iw(g0 + 1, 1)

    def body(g2, carry):
        for p in (0, 1):
            g = g0 + 2 * g2 + p
            cur, nxt = p, 1 - p

            @pl.when(g - g0 < NG - 1)
            def _():
                fire(nxt)

            @pl.when(g - g0 >= 2)
            def _():
                pltpu.make_async_copy(
                    out_v.at[cur], out_hbm.at[pl.ds(base0, G)], osem).wait()

            compute(g, cur)

            @pl.when(g - g0 < NG - 2)
            def _():
                load_iw(g + 2, cur)
        return carry

    lax.fori_loop(0, NG // 2, body, 0)
    for p in (0, 1):
        pltpu.make_async_copy(out_v.at[p], out_hbm.at[pl.ds(base0, G)],
                              osem).wait()


def kernel(batch_hnym, batch_hnym_weights, table):
    idx = batch_hnym.reshape(-1).astype(jnp.int32)
    w = batch_hnym_weights.reshape(-1)
    return _embed_sum(idx, w, table)
